# baseline (speedup 1.0000x reference)
import functools

import jax
import jax.numpy as jnp
from jax import lax
from jax.experimental import pallas as pl
from jax.experimental.pallas import tpu as pltpu
from jax.experimental.pallas import tpu_sc as plsc

_N = 100000
_D = 128
_H = _D // 2
_NW = 32
_B = 128
_BP = _B // 2
_CHUNKS = 25
_NPAD = _NW * _B * _CHUNKS
_NPAD2 = _NPAD // 2

_HIMASK = -65536
_RNE = 0x7FFF


_BM = 1024


def _pack_tc(t):
    e = lax.bitcast_convert_type(t[:, :_H].astype(jnp.bfloat16), jnp.uint16)
    o = lax.bitcast_convert_type(t[:, _H:].astype(jnp.bfloat16), jnp.uint16)
    w = e.astype(jnp.int32) | (o.astype(jnp.int32) << 16)
    w3 = w.reshape(t.shape[0] // 2, 2, _H)
    return jnp.concatenate([w3[:, 0, :], w3[:, 1, :]], axis=1)


def _unpack_tc(w2):
    w = jnp.stack([w2[:, :_H], w2[:, _H:]], axis=1).reshape(w2.shape[0] * 2, _H)
    lo = lax.bitcast_convert_type(w << 16, jnp.float32)
    hi = lax.bitcast_convert_type(w & _HIMASK, jnp.float32)
    return jnp.concatenate([lo, hi], axis=1)


def _mm_tables(x, wc, b):
    y = jnp.dot(x, wc, preferred_element_type=jnp.float32)
    outs = []
    for k in range(5):
        t = y[:, k * _D:(k + 1) * _D]
        if k == 0:
            t = t + b
        outs.append(_pack_tc(t))
    return outs


def _mm_body_f32(x_ref, wc_ref, b_ref, o0, o1, o2, o3, o4):
    outs = _mm_tables(x_ref[...], wc_ref[...], b_ref[...])
    for o, t in zip((o0, o1, o2, o3, o4), outs):
        o[...] = t


def _mm_body_packed(h_ref, wc_ref, b_ref, o0, o1, o2, o3, o4):
    outs = _mm_tables(_unpack_tc(h_ref[...]), wc_ref[...], b_ref[...])
    for o, t in zip((o0, o1, o2, o3, o4), outs):
        o[...] = t


def _tc_tables(xp, wc, b, packed_input):
    grid = _NPAD // _BM
    out_sd = jax.ShapeDtypeStruct((_NPAD2, _D), jnp.int32)
    obs = pl.BlockSpec((_BM // 2, _D), lambda i: (i, 0))
    in_spec = (pl.BlockSpec((_BM // 2, _D), lambda i: (i, 0)) if packed_input
               else pl.BlockSpec((_BM, _D), lambda i: (i, 0)))
    return pl.pallas_call(
        _mm_body_packed if packed_input else _mm_body_f32,
        grid=(grid,),
        in_specs=[
            in_spec,
            pl.BlockSpec((_D, 5 * _D), lambda i: (0, 0)),
            pl.BlockSpec((1, _D), lambda i: (0, 0)),
        ],
        out_specs=[obs, obs, obs, obs, obs],
        out_shape=[out_sd] * 5,
    )(xp, wc, b)


def _lohi(w):
    return (plsc.bitcast(w << 16, jnp.float32),
            plsc.bitcast(w & _HIMASK, jnp.float32))


def _repack(lo, hi):
    lb = plsc.bitcast(lo, jnp.int32)
    hb = plsc.bitcast(hi, jnp.int32)
    lr = lb + _RNE + (jnp.right_shift(lb, 16) & 1)
    hr = hb + _RNE + (jnp.right_shift(hb, 16) & 1)
    return (jnp.right_shift(lr, 16) & 0xFFFF) | (hr & _HIMASK)


def _sc_chunks(refs, final):
    if final:
        (y0_hbm, y1_hbm, y2_hbm, y3_hbm, y4_hbm,
         i0_hbm, i1_hbm, i2_hbm, i3_hbm,
         out_hbm,
         x0_v, x1_v, x2_v, x3_v,
         r0_v, r1_v, r2_v, r3_v,
         acc_v, g0_v, g1_v, g2_v, g3_v, f_v,
         s0, s1, s2, s3) = refs
    else:
        (y0_hbm, y1_hbm, y2_hbm, y3_hbm, y4_hbm,
         i0_hbm, i1_hbm, i2_hbm, i3_hbm,
         out_hbm,
         x0_v, x1_v, x2_v, x3_v,
         r0_v, r1_v, r2_v, r3_v,
         acc_v, g0_v, g1_v, g2_v, g3_v,
         s0, s1, s2, s3) = refs
        f_v = None
    ih = (i0_hbm, i1_hbm, i2_hbm, i3_hbm)
    xv = (x0_v, x1_v, x2_v, x3_v)
    rv = (r0_v, r1_v, r2_v, r3_v)
    gv = (g0_v, g1_v, g2_v, g3_v)
    tbl = (y1_hbm, y2_hbm, y3_hbm, y4_hbm)
    sems = (s0, s1, s2, s3)
    wid = lax.axis_index("s") * 2 + lax.axis_index("c")
    tb0 = wid * (_CHUNKS * _B)
    pb0 = wid * (_CHUNKS * _BP)
    iota = lax.broadcasted_iota(jnp.int32, (16,), 0)
    half = iota & 1
    pair = jnp.right_shift(iota, 1)

    def chunk_body(ci, carry):
        tb = tb0 + ci * _B
        pb = pb0 + ci * _BP
        for k in range(4):
            pltpu.sync_copy(ih[k].at[pl.ds(tb, _B)], xv[k])
        for k in range(4):
            for s8 in range(_B // 16):
                sl = pl.ds(s8 * 16, 16)
                rv[k][sl] = jnp.right_shift(xv[k][sl], 1)
        descs = [pltpu.async_copy(tbl[k].at[rv[k]], gv[k], sems[k])
                 for k in range(4)]
        pltpu.sync_copy(y0_hbm.at[pl.ds(pb, _BP)], acc_v)
        for d in descs:
            d.wait()

        def group_body(g16, gcarry):
            r0 = g16 * 16
            rown = r0 + iota
            racc = r0 // 2 + pair
            cacc = half * _H
            cns = [(xv[k][pl.ds(r0, 16)] & 1) * _H for k in range(4)]

            def word_body(c, wcarry):
                ws = plsc.load_gather(acc_v, [racc, cacc + c])
                lo, hi = _lohi(ws)
                for k in range(4):
                    wk = plsc.load_gather(gv[k], [rown, cns[k] + c])
                    lk, hk = _lohi(wk)
                    lo = lo + lk
                    hi = hi + hk
                lo = jnp.maximum(lo, 0.0)
                hi = jnp.maximum(hi, 0.0)
                if final:
                    cvec = iota * 0 + c
                    plsc.store_scatter(f_v, [rown, cvec], lo)
                    plsc.store_scatter(f_v, [rown, cvec + _H], hi)
                else:
                    plsc.store_scatter(acc_v, [racc, cacc + c],
                                       _repack(lo, hi))
                return wcarry

            lax.fori_loop(0, _H, word_body, 0)
            return gcarry

        lax.fori_loop(0, _B // 16, group_body, 0)
        if final:
            pltpu.sync_copy(f_v, out_hbm.at[pl.ds(tb, _B)])
        else:
            pltpu.sync_copy(acc_v, out_hbm.at[pl.ds(pb, _BP)])
        return carry

    lax.fori_loop(0, _CHUNKS, chunk_body, 0)


_SC_SCRATCH = [
    pltpu.VMEM((_B,), jnp.int32),
    pltpu.VMEM((_B,), jnp.int32),
    pltpu.VMEM((_B,), jnp.int32),
    pltpu.VMEM((_B,), jnp.int32),
    pltpu.VMEM((_B,), jnp.int32),
    pltpu.VMEM((_B,), jnp.int32),
    pltpu.VMEM((_B,), jnp.int32),
    pltpu.VMEM((_B,), jnp.int32),
    pltpu.VMEM((_BP, _D), jnp.int32),
    pltpu.VMEM((_B, _D), jnp.int32),
    pltpu.VMEM((_B, _D), jnp.int32),
    pltpu.VMEM((_B, _D), jnp.int32),
    pltpu.VMEM((_B, _D), jnp.int32),
]
_SC_SEMS = [
    pltpu.SemaphoreType.DMA,
    pltpu.SemaphoreType.DMA,
    pltpu.SemaphoreType.DMA,
    pltpu.SemaphoreType.DMA,
]
_SC_PARAMS = pltpu.CompilerParams(needs_layout_passes=False)


@functools.cache
def _sc_mid_kernel():
    return pl.kernel(
        lambda *refs: _sc_chunks(refs, final=False),
        mesh=plsc.VectorSubcoreMesh(core_axis_name="c", subcore_axis_name="s"),
        out_type=jax.ShapeDtypeStruct((_NPAD2, _D), jnp.int32),
        scratch_types=_SC_SCRATCH + _SC_SEMS,
        compiler_params=_SC_PARAMS,
    )


@functools.cache
def _sc_final_kernel():
    return pl.kernel(
        lambda *refs: _sc_chunks(refs, final=True),
        mesh=plsc.VectorSubcoreMesh(core_axis_name="c", subcore_axis_name="s"),
        out_type=jax.ShapeDtypeStruct((_NPAD, _D), jnp.float32),
        scratch_types=_SC_SCRATCH + [pltpu.VMEM((_B, _D), jnp.float32)]
        + _SC_SEMS,
        compiler_params=_SC_PARAMS,
    )


def kernel(x, neighbors, W0, b0, W1, b1):
    xp = jnp.pad(x, ((0, _NPAD - _N), (0, 0)))
    nb = jnp.pad(neighbors.astype(jnp.int32), ((0, _NPAD - _N), (0, 0)))
    i0 = nb[:, 0]
    i1 = nb[:, 1]
    i2 = nb[:, 2]
    i3 = nb[:, 3]

    def wcat(W):
        return W.reshape(5, _D, _D).transpose(1, 0, 2).reshape(_D, 5 * _D)

    y = _tc_tables(xp, wcat(W0), b0.reshape(1, _D), packed_input=False)
    h1 = _sc_mid_kernel()(*y, i0, i1, i2, i3)
    y = _tc_tables(h1, wcat(W1), b1.reshape(1, _D), packed_input=True)
    out = _sc_final_kernel()(*y, i0, i1, i2, i3)
    return out[:_N]


# speedup vs baseline: 3.0816x; 3.0816x over previous
import functools

import jax
import jax.numpy as jnp
from jax import lax
from jax.experimental import pallas as pl
from jax.experimental.pallas import tpu as pltpu
from jax.experimental.pallas import tpu_sc as plsc

_N = 100000
_D = 128
_NW = 32
_B = 64
_CHUNKS = 50
_NPAD = _NW * _B * _CHUNKS


_BM = 1024


def _mm_body(x_ref, wc_ref, b_ref, o0, o1, o2, o3, o4):
    y = jnp.dot(x_ref[...], wc_ref[...], preferred_element_type=jnp.float32)
    o0[...] = y[:, 0 * _D:1 * _D] + b_ref[...]
    o1[...] = y[:, 1 * _D:2 * _D]
    o2[...] = y[:, 2 * _D:3 * _D]
    o3[...] = y[:, 3 * _D:4 * _D]
    o4[...] = y[:, 4 * _D:5 * _D]


def _tc_tables(xp, wc, b):
    grid = _NPAD // _BM
    out_sd = jax.ShapeDtypeStruct((_NPAD, _D), jnp.float32)
    obs = pl.BlockSpec((_BM, _D), lambda i: (i, 0))
    return pl.pallas_call(
        _mm_body,
        grid=(grid,),
        in_specs=[
            pl.BlockSpec((_BM, _D), lambda i: (i, 0)),
            pl.BlockSpec((_D, 5 * _D), lambda i: (0, 0)),
            pl.BlockSpec((1, _D), lambda i: (0, 0)),
        ],
        out_specs=[obs, obs, obs, obs, obs],
        out_shape=[out_sd, out_sd, out_sd, out_sd, out_sd],
    )(xp, wc, b)


def _sc_body(y0_hbm, y1_hbm, y2_hbm, y3_hbm, y4_hbm,
             i0_hbm, i1_hbm, i2_hbm, i3_hbm,
             out_hbm, *scr):
    sets = []
    for sidx in range(2):
        o = sidx * 9
        sets.append(dict(
            xv=scr[o:o + 4], acc=scr[o + 4], gv=scr[o + 5:o + 9],
            sems=scr[18 + sidx * 5:18 + sidx * 5 + 5],
        ))
    ih = (i0_hbm, i1_hbm, i2_hbm, i3_hbm)
    tbl = (y1_hbm, y2_hbm, y3_hbm, y4_hbm)
    wid = lax.axis_index("s") * 2 + lax.axis_index("c")
    base0 = wid * (_CHUNKS * _B)

    def issue(ci, st):
        base = base0 + ci * _B
        for k in range(4):
            pltpu.sync_copy(ih[k].at[pl.ds(base, _B)], st["xv"][k])
        for k in range(4):
            pltpu.async_copy(tbl[k].at[st["xv"][k]], st["gv"][k],
                             st["sems"][k])
        pltpu.async_copy(y0_hbm.at[pl.ds(base, _B)], st["acc"],
                         st["sems"][4])

    def finish(ci, st):
        base = base0 + ci * _B
        acc_v = st["acc"]
        g0_v, g1_v, g2_v, g3_v = st["gv"]
        for k in range(4):
            pltpu.make_async_copy(tbl[k].at[pl.ds(0, _B)], st["gv"][k],
                                  st["sems"][k]).wait()
        pltpu.make_async_copy(y0_hbm.at[pl.ds(0, _B)], acc_v,
                              st["sems"][4]).wait()

        def row_body(r, rcarry):
            for c in range(_D // 16):
                s = pl.ds(c * 16, 16)
                v = (acc_v[r, s] + g0_v[r, s] + g1_v[r, s]
                     + g2_v[r, s] + g3_v[r, s])
                acc_v[r, s] = jnp.maximum(v, 0.0)
            return rcarry

        lax.fori_loop(0, _B, row_body, 0)
        pltpu.sync_copy(acc_v, out_hbm.at[pl.ds(base, _B)])

    issue(0, sets[0])

    def pair_body(i, carry):
        c0 = 2 * i
        issue(c0 + 1, sets[1])
        finish(c0, sets[0])

        @pl.when(c0 + 2 < _CHUNKS)
        def _():
            issue(c0 + 2, sets[0])

        finish(c0 + 1, sets[1])
        return carry

    lax.fori_loop(0, _CHUNKS // 2, pair_body, 0)


@functools.cache
def _sc_gather_sum_kernel():
    scratch = []
    for _ in range(2):
        scratch += [pltpu.VMEM((_B,), jnp.int32) for _ in range(4)]
        scratch += [pltpu.VMEM((_B, _D), jnp.float32) for _ in range(5)]
    scratch += [pltpu.SemaphoreType.DMA for _ in range(10)]
    return pl.kernel(
        _sc_body,
        mesh=plsc.VectorSubcoreMesh(core_axis_name="c", subcore_axis_name="s"),
        out_type=jax.ShapeDtypeStruct((_NPAD, _D), jnp.float32),
        scratch_types=scratch,
    )


def _sc_gather_sum(*args):
    return _sc_gather_sum_kernel()(*args)


def kernel(x, neighbors, W0, b0, W1, b1):
    xp = jnp.pad(x, ((0, _NPAD - _N), (0, 0)))
    nb = jnp.pad(neighbors.astype(jnp.int32), ((0, _NPAD - _N), (0, 0)))
    i0 = nb[:, 0]
    i1 = nb[:, 1]
    i2 = nb[:, 2]
    i3 = nb[:, 3]

    def wcat(W):
        return W.reshape(5, _D, _D).transpose(1, 0, 2).reshape(_D, 5 * _D)

    y = _tc_tables(xp, wcat(W0), b0.reshape(1, _D))
    h1 = _sc_gather_sum(y[0], y[1], y[2], y[3], y[4], i0, i1, i2, i3)
    y = _tc_tables(h1, wcat(W1), b1.reshape(1, _D))
    h2 = _sc_gather_sum(y[0], y[1], y[2], y[3], y[4], i0, i1, i2, i3)
    return h2[:_N]


# speedup vs baseline: 3.2968x; 1.0698x over previous
import functools

import jax
import jax.numpy as jnp
from jax import lax
from jax.experimental import pallas as pl
from jax.experimental.pallas import tpu as pltpu
from jax.experimental.pallas import tpu_sc as plsc

_N = 100000
_D = 128
_NW = 32
_B = 64
_CHUNKS = 100
_CH_A = 60
_CH_B = 40
_NPAD = 16 * _B * _CHUNKS


_BM = 1024


def _mm_body(x_ref, wc_ref, b_ref, o0, o1, o2, o3, o4):
    y = jnp.dot(x_ref[...], wc_ref[...], preferred_element_type=jnp.float32)
    o0[...] = y[:, 0 * _D:1 * _D] + b_ref[...]
    o1[...] = y[:, 1 * _D:2 * _D]
    o2[...] = y[:, 2 * _D:3 * _D]
    o3[...] = y[:, 3 * _D:4 * _D]
    o4[...] = y[:, 4 * _D:5 * _D]


def _tc_tables(xp, wc, b):
    grid = _NPAD // _BM
    out_sd = jax.ShapeDtypeStruct((_NPAD, _D), jnp.float32)
    obs = pl.BlockSpec((_BM, _D), lambda i: (i, 0))
    return pl.pallas_call(
        _mm_body,
        grid=(grid,),
        in_specs=[
            pl.BlockSpec((_BM, _D), lambda i: (i, 0)),
            pl.BlockSpec((_D, 5 * _D), lambda i: (0, 0)),
            pl.BlockSpec((1, _D), lambda i: (0, 0)),
        ],
        out_specs=[obs, obs, obs, obs, obs],
        out_shape=[out_sd, out_sd, out_sd, out_sd, out_sd],
    )(xp, wc, b)


def _sc_body(y0_hbm, y1_hbm, y2_hbm, y3_hbm, y4_hbm,
             i0_hbm, i1_hbm, i2_hbm, i3_hbm,
             out_hbm, *scr):
    sets = []
    for sidx in range(2):
        o = sidx * 9
        sets.append(dict(
            xv=scr[o:o + 4], acc=scr[o + 4], gv=scr[o + 5:o + 9],
            sems=scr[18 + sidx * 5:18 + sidx * 5 + 5],
        ))
    ih = (i0_hbm, i1_hbm, i2_hbm, i3_hbm)
    tbl = (y1_hbm, y2_hbm, y3_hbm, y4_hbm)
    cc = lax.axis_index("c")
    ss = lax.axis_index("s")
    nch = jnp.where(cc == 0, _CH_A, _CH_B)
    base0 = jnp.where(cc == 0, ss * _CH_A, 16 * _CH_A + ss * _CH_B) * _B

    def issue(ci, st):
        base = base0 + ci * _B
        for k in range(4):
            pltpu.sync_copy(ih[k].at[pl.ds(base, _B)], st["xv"][k])
        for k in range(4):
            pltpu.async_copy(tbl[k].at[st["xv"][k]], st["gv"][k],
                             st["sems"][k])
        pltpu.async_copy(y0_hbm.at[pl.ds(base, _B)], st["acc"],
                         st["sems"][4])

    def finish(ci, st):
        base = base0 + ci * _B
        acc_v = st["acc"]
        g0_v, g1_v, g2_v, g3_v = st["gv"]
        for k in range(4):
            pltpu.make_async_copy(tbl[k].at[pl.ds(0, _B)], st["gv"][k],
                                  st["sems"][k]).wait()
        pltpu.make_async_copy(y0_hbm.at[pl.ds(0, _B)], acc_v,
                              st["sems"][4]).wait()

        def row_body(r, rcarry):
            for c in range(_D // 16):
                s = pl.ds(c * 16, 16)
                v = (acc_v[r, s] + g0_v[r, s] + g1_v[r, s]
                     + g2_v[r, s] + g3_v[r, s])
                acc_v[r, s] = jnp.maximum(v, 0.0)
            return rcarry

        lax.fori_loop(0, _B, row_body, 0)
        pltpu.sync_copy(acc_v, out_hbm.at[pl.ds(base, _B)])

    issue(0, sets[0])

    def pair_body(i, carry):
        c0 = 2 * i
        issue(c0 + 1, sets[1])
        finish(c0, sets[0])

        @pl.when(c0 + 2 < nch)
        def _():
            issue(c0 + 2, sets[0])

        finish(c0 + 1, sets[1])
        return carry

    lax.fori_loop(0, nch // 2, pair_body, 0)


@functools.cache
def _sc_gather_sum_kernel():
    scratch = []
    for _ in range(2):
        scratch += [pltpu.VMEM((_B,), jnp.int32) for _ in range(4)]
        scratch += [pltpu.VMEM((_B, _D), jnp.float32) for _ in range(5)]
    scratch += [pltpu.SemaphoreType.DMA for _ in range(10)]
    return pl.kernel(
        _sc_body,
        mesh=plsc.VectorSubcoreMesh(core_axis_name="c", subcore_axis_name="s"),
        out_type=jax.ShapeDtypeStruct((_NPAD, _D), jnp.float32),
        scratch_types=scratch,
    )


def _sc_gather_sum(*args):
    return _sc_gather_sum_kernel()(*args)


def kernel(x, neighbors, W0, b0, W1, b1):
    xp = jnp.pad(x, ((0, _NPAD - _N), (0, 0)))
    nb = jnp.pad(neighbors.astype(jnp.int32), ((0, _NPAD - _N), (0, 0)))
    i0 = nb[:, 0]
    i1 = nb[:, 1]
    i2 = nb[:, 2]
    i3 = nb[:, 3]

    def wcat(W):
        return W.reshape(5, _D, _D).transpose(1, 0, 2).reshape(_D, 5 * _D)

    y = _tc_tables(xp, wcat(W0), b0.reshape(1, _D))
    h1 = _sc_gather_sum(y[0], y[1], y[2], y[3], y[4], i0, i1, i2, i3)
    y = _tc_tables(h1, wcat(W1), b1.reshape(1, _D))
    h2 = _sc_gather_sum(y[0], y[1], y[2], y[3], y[4], i0, i1, i2, i3)
    return h2[:_N]
